# baseline (speedup 1.0000x reference)
import jax
import jax.numpy as jnp
from jax import lax
from jax.experimental import pallas as pl
from jax.experimental.pallas import tpu as pltpu
from jax.experimental.pallas import tpu_sc as plsc

_NC = 2
_NS = 16
_BN = 1024
_S = 4096


def _gather_body(idx_hbm, cv_hbm, row_hbm, idx_s):
    first = lax.axis_index("c") == 0

    @pl.when(first)
    def _():
        pltpu.sync_copy(idx_hbm, idx_s)
        s = idx_s[0]
        pltpu.sync_copy(cv_hbm.at[pl.ds(s, 1), :], row_hbm)


def _head_body(idx_ref, h_ref, cv_ref, o_ref):
    o_ref[...] = h_ref[...] + cv_ref[0]


def _tail_body(acc_ref, h_ref, row_ref, o_ref):
    del acc_ref
    o_ref[...] = h_ref[...] + row_ref[...]


def kernel(hidden_states, affective_state_index, control_vectors):
    n, d = hidden_states.shape
    k = control_vectors.shape[0]
    idx = jnp.asarray(affective_state_index, jnp.int32).reshape(1)

    row = pl.kernel(
        _gather_body,
        out_type=jax.ShapeDtypeStruct((1, d), control_vectors.dtype),
        mesh=plsc.ScalarSubcoreMesh(axis_name="c", num_cores=_NC),
        scratch_types=[
            pltpu.SMEM((1,), jnp.int32),
        ],
    )(idx, control_vectors)

    cv3 = control_vectors.reshape(k, 1, d)
    head = pl.pallas_call(
        _head_body,
        grid_spec=pltpu.PrefetchScalarGridSpec(
            num_scalar_prefetch=1,
            grid=(_S // _BN,),
            in_specs=[
                pl.BlockSpec((_BN, d), lambda i, idx_ref: (i, 0)),
                pl.BlockSpec((1, 1, d), lambda i, idx_ref: (idx_ref[0], 0, 0)),
            ],
            out_specs=pl.BlockSpec((_BN, d), lambda i, idx_ref: (i, 0)),
        ),
        out_shape=jax.ShapeDtypeStruct((n, d), hidden_states.dtype),
    )(idx, hidden_states, cv3)

    off = _S // _BN
    return pl.pallas_call(
        _tail_body,
        grid=((n - _S) // _BN,),
        in_specs=[
            pl.BlockSpec(memory_space=pl.ANY),
            pl.BlockSpec((_BN, d), lambda i: (i + off, 0)),
            pl.BlockSpec((1, d), lambda i: (0, 0)),
        ],
        out_specs=pl.BlockSpec((_BN, d), lambda i: (i + off, 0)),
        out_shape=jax.ShapeDtypeStruct((n, d), hidden_states.dtype),
        input_output_aliases={0: 0},
    )(head, hidden_states, row)
